# baseline (speedup 1.0000x reference)
import functools

import jax
import jax.numpy as jnp
import numpy as np
from jax.experimental import pallas as pl
from jax.experimental.pallas import tpu as pltpu

_NUM_NODES = 9
_HIDDEN = 128
_B_TILE = 512
_DEPTH = 3


def _neighbor_lists(num_nodes):
    doubling = np.zeros((num_nodes, num_nodes), dtype=np.float32)
    for src, dst in [(0, 1), (1, 3), (3, 7), (7, 6), (6, 4), (4, 0)]:
        doubling[dst, src] = 1
    comp = np.zeros((num_nodes, num_nodes), dtype=np.float32)
    for a, b in [(0, 7), (1, 6), (3, 4), (2, 5)]:
        comp[a, b] = comp[b, a] = 1
    central = np.zeros((num_nodes, num_nodes), dtype=np.float32)
    for i in range(8):
        central[i, 8] = central[8, i] = 1
    neigh = []
    for i in range(num_nodes):
        lst = []
        for adj in (doubling, comp, central):
            lst.extend(int(j) for j in np.nonzero(adj[i])[0])
        neigh.append(lst)
    return neigh

_NEIGH = _neighbor_lists(_NUM_NODES)


def _body(nf_hbm, wto_ref, bto_ref, wfrom_ref, bfrom_ref, curv_ref, mwt_ref,
          out_hbm, in_buf, out_buf, in_sem, out_sem):
    n_grid = pl.num_programs(0)
    k = pl.program_id(0)

    def in_copy(step, slot):
        return pltpu.make_async_copy(
            nf_hbm.at[pl.ds(step * _B_TILE, _B_TILE)],
            in_buf.at[slot],
            in_sem.at[slot])

    def out_copy(step, slot):
        return pltpu.make_async_copy(
            out_buf.at[slot],
            out_hbm.at[pl.ds(step * _B_TILE, _B_TILE)],
            out_sem.at[slot])

    slot = jax.lax.rem(k, _DEPTH)
    nslot = jax.lax.rem(k + 1, _DEPTH)

    @pl.when(k == 0)
    def _prologue():
        in_copy(k, slot).start()
        in_copy(k + 1, nslot).start()

    @pl.when(k + 2 < n_grid)
    def _prefetch():
        in_copy(k + 2, jax.lax.rem(k + 2, _DEPTH)).start()

    in_copy(k, slot).wait()

    c = jnp.abs(curv_ref[0, 0])
    bto = bto_ref[...]
    bfrom = bfrom_ref[...]

    hyp = []
    x2 = []
    for i in range(_NUM_NODES):
        x = in_buf[slot, :, i, :]
        p = jax.lax.dot_general(wto_ref[...], x, (((1,), (1,)), ((), ())),
                                preferred_element_type=jnp.float32) + bto
        n2 = jnp.sum(p * p, axis=0, keepdims=True)
        n = jnp.sqrt(n2)
        scale = jnp.tanh(n) / (n + 1e-08)
        hyp.append(p * scale)
        x2.append(n2 * scale * scale)

    @pl.when(k >= _DEPTH)
    def _drain_prev():
        out_copy(k - _DEPTH, slot).wait()

    for i in range(_NUM_NODES):
        acc = hyp[i]
        a2 = x2[i]
        for j in _NEIGH[i]:
            w = mwt_ref[:, pl.ds(i * _NUM_NODES + j, 1)]
            w2 = jnp.sum(w * w, axis=0, keepdims=True)
            xw = jnp.sum(hyp[j] * w, axis=0, keepdims=True)
            r = 1.0 / (1.0 + 2.0 * c * xw + (c * c) * x2[j] * w2 + 1e-08)
            ca = (1.0 + 2.0 * c * xw + c * w2) * r
            cb = (1.0 - c * x2[j]) * r
            t = ca * hyp[j] + cb * w
            t2 = ca * ca * x2[j] + 2.0 * ca * cb * xw + cb * cb * w2
            at = jnp.sum(acc * t, axis=0, keepdims=True)
            rr = 1.0 / (1.0 + 2.0 * c * at + (c * c) * a2 * t2 + 1e-08)
            ga = (1.0 + 2.0 * c * at + c * t2) * rr
            gb = (1.0 - c * a2) * rr
            acc = ga * acc + gb * t
            a2 = ga * ga * a2 + 2.0 * ga * gb * at + gb * gb * t2
        out_buf[slot, :, i, :] = jax.lax.dot_general(
            acc, wfrom_ref[...], (((0,), (1,)), ((), ())),
            preferred_element_type=jnp.float32) + bfrom

    out_copy(k, slot).start()

    @pl.when(k == n_grid - 1)
    def _epilogue():
        for d in range(_DEPTH):
            @pl.when(k >= d)
            def _():
                out_copy(k - d, jax.lax.rem(k - d, _DEPTH)).wait()


@functools.partial(jax.jit, static_argnames=("interpret",))
def kernel(node_features, W_to, b_to, W_from, b_from, curvature,
           mobius_weights, interpret=False):
    batch = node_features.shape[0]
    grid = batch // _B_TILE

    full = lambda shape: pl.BlockSpec(shape, lambda b: (0,) * len(shape))
    out = pl.pallas_call(
        _body,
        grid=(grid,),
        in_specs=[pl.BlockSpec(memory_space=pltpu.MemorySpace.HBM)] + [
            full((_HIDDEN, _HIDDEN)),
            full((_HIDDEN, 1)),
            full((_HIDDEN, _HIDDEN)),
            full((1, _HIDDEN)),
            full((1, 1)),
            full((_HIDDEN, _NUM_NODES * _NUM_NODES)),
        ],
        out_specs=pl.BlockSpec(memory_space=pltpu.MemorySpace.HBM),
        out_shape=jax.ShapeDtypeStruct((batch, _NUM_NODES, _HIDDEN),
                                       jnp.float32),
        scratch_shapes=(
            [pltpu.VMEM((_DEPTH, _B_TILE, _NUM_NODES, _HIDDEN), jnp.float32),
             pltpu.VMEM((_DEPTH, _B_TILE, _NUM_NODES, _HIDDEN), jnp.float32),
             pltpu.SemaphoreType.DMA((_DEPTH,)),
             pltpu.SemaphoreType.DMA((_DEPTH,))]
        ),
        interpret=interpret,
    )(
        node_features,
        W_to,
        b_to.reshape(_HIDDEN, 1),
        W_from,
        b_from.reshape(1, _HIDDEN),
        jnp.asarray(curvature, jnp.float32).reshape(1, 1),
        mobius_weights.reshape(_NUM_NODES * _NUM_NODES, _HIDDEN).T,
    )
    return out


# speedup vs baseline: 1.0007x; 1.0007x over previous
import functools

import jax
import jax.numpy as jnp
import numpy as np
from jax.experimental import pallas as pl
from jax.experimental.pallas import tpu as pltpu

_NUM_NODES = 9
_HIDDEN = 128
_B_TILE = 512
_DEPTH = 3


def _neighbor_lists(num_nodes):
    doubling = np.zeros((num_nodes, num_nodes), dtype=np.float32)
    for src, dst in [(0, 1), (1, 3), (3, 7), (7, 6), (6, 4), (4, 0)]:
        doubling[dst, src] = 1
    comp = np.zeros((num_nodes, num_nodes), dtype=np.float32)
    for a, b in [(0, 7), (1, 6), (3, 4), (2, 5)]:
        comp[a, b] = comp[b, a] = 1
    central = np.zeros((num_nodes, num_nodes), dtype=np.float32)
    for i in range(8):
        central[i, 8] = central[8, i] = 1
    neigh = []
    for i in range(num_nodes):
        lst = []
        for adj in (doubling, comp, central):
            lst.extend(int(j) for j in np.nonzero(adj[i])[0])
        neigh.append(lst)
    return neigh

_NEIGH = _neighbor_lists(_NUM_NODES)
_EDGES = [(i, j) for i in range(_NUM_NODES) for j in _NEIGH[i]]
_NUM_EDGES = len(_EDGES)
_EPAD = 32

_PAIRS = [(a, b) for a in range(_NUM_NODES) for b in range(a, _NUM_NODES)]
_NUM_CHUNKS = len(_PAIRS) // _NUM_NODES


def _body(nf_hbm, wto_ref, bto_ref, wfrom_ref, bfrom_ref, curv_ref, mwt_ref,
          wstack_ref, onesblk_ref, out_hbm, in_buf, out_buf, s_ref, p_ref,
          in_sem, out_sem):
    n_grid = pl.num_programs(0)
    k = pl.program_id(0)

    def in_copy(step, slot):
        return pltpu.make_async_copy(
            nf_hbm.at[pl.ds(step * _B_TILE, _B_TILE)],
            in_buf.at[slot],
            in_sem.at[slot])

    def out_copy(step, slot):
        return pltpu.make_async_copy(
            out_buf.at[slot],
            out_hbm.at[pl.ds(step * _B_TILE, _B_TILE)],
            out_sem.at[slot])

    slot = jax.lax.rem(k, _DEPTH)
    nslot = jax.lax.rem(k + 1, _DEPTH)

    @pl.when(k == 0)
    def _prologue():
        in_copy(k, slot).start()
        in_copy(k + 1, nslot).start()

    @pl.when(k + 2 < n_grid)
    def _prefetch():
        in_copy(k + 2, jax.lax.rem(k + 2, _DEPTH)).start()

    in_copy(k, slot).wait()

    c = jnp.abs(curv_ref[0, 0])
    bto = bto_ref[...]
    bfrom = bfrom_ref[...]
    wstack = wstack_ref[...]

    for a in range(_NUM_NODES):
        x = in_buf[slot, :, a, :]
        p = jax.lax.dot_general(wto_ref[...], x, (((1,), (1,)), ((), ())),
                                preferred_element_type=jnp.float32) + bto
        s_ref[pl.ds(a * _HIDDEN, _HIDDEN), :] = p

    wh = []
    for a in range(_NUM_NODES):
        pa = s_ref[pl.ds(a * _HIDDEN, _HIDDEN), :]
        wh.append(jax.lax.dot_general(
            wstack, pa, (((1,), (0,)), ((), ())),
            preferred_element_type=jnp.float32))
    ww = jax.lax.dot_general(wstack, wstack, (((1,), (1,)), ((), ())),
                             preferred_element_type=jnp.float32)

    gchunks = []
    for cidx in range(_NUM_CHUNKS):
        for r in range(_NUM_NODES):
            a, b = _PAIRS[cidx * _NUM_NODES + r]
            pa = s_ref[pl.ds(a * _HIDDEN, _HIDDEN), :]
            pb = s_ref[pl.ds(b * _HIDDEN, _HIDDEN), :]
            p_ref[pl.ds(r * _HIDDEN, _HIDDEN), :] = pa * pb
        gchunks.append(jax.lax.dot_general(
            onesblk_ref[...], p_ref[...], (((1,), (0,)), ((), ())),
            preferred_element_type=jnp.float32))

    def gram(a, b):
        idx = _PAIRS.index((min(a, b), max(a, b)))
        r = idx % _NUM_NODES
        return gchunks[idx // _NUM_NODES][r:r + 1, :]

    def whd(e, a):
        return wh[a][e:e + 1, :]

    def wwd(e, f):
        return ww[e:e + 1, f:f + 1]

    sc = []
    x2 = []
    for a in range(_NUM_NODES):
        n2 = gram(a, a)
        n = jnp.sqrt(n2)
        s = jnp.tanh(n) / (n + 1e-08)
        sc.append(s)
        x2.append(n2 * s * s)

    @pl.when(k >= _DEPTH)
    def _drain_prev():
        out_copy(k - _DEPTH, slot).wait()

    eid = 0
    for i in range(_NUM_NODES):
        ejs = [(eid + kk, j) for kk, j in enumerate(_NEIGH[i])]
        eid += len(ejs)
        d = len(ejs)
        D = {j: sc[i] * sc[j] * gram(i, j) for _, j in ejs}
        E = {e: sc[i] * whd(e, i) for e, _ in ejs}
        a2 = x2[i]
        coeffs = []
        for kk, (e, j) in enumerate(ejs):
            w2 = wwd(e, e)
            xw = sc[j] * whd(e, j)
            r = 1.0 / (1.0 + 2.0 * c * xw + (c * c) * x2[j] * w2 + 1e-08)
            ca = (1.0 + 2.0 * c * xw + c * w2) * r
            cb = (1.0 - c * x2[j]) * r
            t2 = ca * ca * x2[j] + 2.0 * ca * cb * xw + cb * cb * w2
            at = ca * D[j] + cb * E[e]
            rr = 1.0 / (1.0 + 2.0 * c * at + (c * c) * a2 * t2 + 1e-08)
            ga = (1.0 + 2.0 * c * at + c * t2) * rr
            gb = (1.0 - c * a2) * rr
            a2 = ga * ga * a2 + 2.0 * ga * gb * at + gb * gb * t2
            for e2, j2 in ejs[kk + 1:]:
                tdn = ca * sc[j] * sc[j2] * gram(j, j2) + cb * sc[j2] * whd(e, j2)
                D[j2] = ga * D[j2] + gb * tdn
                tdw = ca * sc[j] * whd(e2, j) + cb * wwd(e, e2)
                E[e2] = ga * E[e2] + gb * tdw
            coeffs.append((e, j, ca, cb, ga, gb))
        suf = [None] * d
        run = None
        for kk in range(d - 1, -1, -1):
            suf[kk] = run
            ga_k = coeffs[kk][4]
            run = ga_k if run is None else ga_k * run
        lam = run
        acc = (lam * sc[i]) * s_ref[pl.ds(i * _HIDDEN, _HIDDEN), :]
        for kk, (e, j, ca, cb, ga, gb) in enumerate(coeffs):
            s_k = gb if suf[kk] is None else gb * suf[kk]
            beta = s_k * ca * sc[j]
            gamma = s_k * cb
            i_e, j_e = _EDGES[e]
            wcol = mwt_ref[:, pl.ds(i_e * _NUM_NODES + j_e, 1)]
            acc = acc + beta * s_ref[pl.ds(j * _HIDDEN, _HIDDEN), :] \
                      + gamma * wcol
        out_buf[slot, :, i, :] = jax.lax.dot_general(
            acc, wfrom_ref[...], (((0,), (1,)), ((), ())),
            preferred_element_type=jnp.float32) + bfrom

    out_copy(k, slot).start()

    @pl.when(k == n_grid - 1)
    def _epilogue():
        for dd in range(_DEPTH):
            @pl.when(k >= dd)
            def _():
                out_copy(k - dd, jax.lax.rem(k - dd, _DEPTH)).wait()


@functools.partial(jax.jit, static_argnames=("interpret",))
def kernel(node_features, W_to, b_to, W_from, b_from, curvature,
           mobius_weights, interpret=False):
    batch = node_features.shape[0]
    grid = batch // _B_TILE

    wrows = [mobius_weights[i, j] for i, j in _EDGES]
    wrows += [jnp.zeros((_HIDDEN,), jnp.float32)] * (_EPAD - _NUM_EDGES)
    wstack = jnp.stack(wrows, axis=0)

    onesblk = np.zeros((16, _NUM_NODES * _HIDDEN), dtype=np.float32)
    for r in range(_NUM_NODES):
        onesblk[r, r * _HIDDEN:(r + 1) * _HIDDEN] = 1.0

    full = lambda shape: pl.BlockSpec(shape, lambda b: (0,) * len(shape))
    out = pl.pallas_call(
        _body,
        grid=(grid,),
        in_specs=[pl.BlockSpec(memory_space=pltpu.MemorySpace.HBM)] + [
            full((_HIDDEN, _HIDDEN)),
            full((_HIDDEN, 1)),
            full((_HIDDEN, _HIDDEN)),
            full((1, _HIDDEN)),
            full((1, 1)),
            full((_HIDDEN, _NUM_NODES * _NUM_NODES)),
            full((_EPAD, _HIDDEN)),
            full((16, _NUM_NODES * _HIDDEN)),
        ],
        out_specs=pl.BlockSpec(memory_space=pltpu.MemorySpace.HBM),
        out_shape=jax.ShapeDtypeStruct((batch, _NUM_NODES, _HIDDEN),
                                       jnp.float32),
        scratch_shapes=(
            [pltpu.VMEM((_DEPTH, _B_TILE, _NUM_NODES, _HIDDEN), jnp.float32),
             pltpu.VMEM((_DEPTH, _B_TILE, _NUM_NODES, _HIDDEN), jnp.float32),
             pltpu.VMEM((_NUM_NODES * _HIDDEN, _B_TILE), jnp.float32),
             pltpu.VMEM((_NUM_NODES * _HIDDEN, _B_TILE), jnp.float32),
             pltpu.SemaphoreType.DMA((_DEPTH,)),
             pltpu.SemaphoreType.DMA((_DEPTH,))]
        ),
        interpret=interpret,
    )(
        node_features,
        W_to,
        b_to.reshape(_HIDDEN, 1),
        W_from,
        b_from.reshape(1, _HIDDEN),
        jnp.asarray(curvature, jnp.float32).reshape(1, 1),
        mobius_weights.reshape(_NUM_NODES * _NUM_NODES, _HIDDEN).T,
        wstack,
        jnp.asarray(onesblk),
    )
    return out


# speedup vs baseline: 1.0302x; 1.0295x over previous
import functools

import jax
import jax.numpy as jnp
import numpy as np
from jax.experimental import pallas as pl
from jax.experimental.pallas import tpu as pltpu

_NUM_NODES = 9
_HIDDEN = 128
_B_TILE = 512


def _neighbor_lists(num_nodes):
    doubling = np.zeros((num_nodes, num_nodes), dtype=np.float32)
    for src, dst in [(0, 1), (1, 3), (3, 7), (7, 6), (6, 4), (4, 0)]:
        doubling[dst, src] = 1
    comp = np.zeros((num_nodes, num_nodes), dtype=np.float32)
    for a, b in [(0, 7), (1, 6), (3, 4), (2, 5)]:
        comp[a, b] = comp[b, a] = 1
    central = np.zeros((num_nodes, num_nodes), dtype=np.float32)
    for i in range(8):
        central[i, 8] = central[8, i] = 1
    neigh = []
    for i in range(num_nodes):
        lst = []
        for adj in (doubling, comp, central):
            lst.extend(int(j) for j in np.nonzero(adj[i])[0])
        neigh.append(lst)
    return neigh

_NEIGH = _neighbor_lists(_NUM_NODES)
_EDGES = [(i, j) for i in range(_NUM_NODES) for j in _NEIGH[i]]
_NUM_EDGES = len(_EDGES)
_EPAD = 32

_PAIRS = [(a, b) for a in range(_NUM_NODES) for b in range(a, _NUM_NODES)]
_NUM_CHUNKS = len(_PAIRS) // _NUM_NODES


def _body(nf_ref, wto_ref, bto_ref, wfrom_ref, bfrom_ref, curv_ref, mwt_ref,
          wstack_ref, onesblk_ref, out_ref, s_ref, p_ref):
    c = jnp.abs(curv_ref[0, 0])
    bto = bto_ref[...]
    bfrom = bfrom_ref[...]
    wstack = wstack_ref[...]

    for a in range(_NUM_NODES):
        x = nf_ref[:, a, :]
        p = jax.lax.dot_general(wto_ref[...], x, (((1,), (1,)), ((), ())),
                                preferred_element_type=jnp.float32) + bto
        s_ref[pl.ds(a * _HIDDEN, _HIDDEN), :] = p

    wh = []
    for a in range(_NUM_NODES):
        pa = s_ref[pl.ds(a * _HIDDEN, _HIDDEN), :]
        wh.append(jax.lax.dot_general(
            wstack, pa, (((1,), (0,)), ((), ())),
            preferred_element_type=jnp.float32))
    ww = jax.lax.dot_general(wstack, wstack, (((1,), (1,)), ((), ())),
                             preferred_element_type=jnp.float32)

    gchunks = []
    for cidx in range(_NUM_CHUNKS):
        for r in range(_NUM_NODES):
            a, b = _PAIRS[cidx * _NUM_NODES + r]
            pa = s_ref[pl.ds(a * _HIDDEN, _HIDDEN), :]
            pb = s_ref[pl.ds(b * _HIDDEN, _HIDDEN), :]
            p_ref[pl.ds(r * _HIDDEN, _HIDDEN), :] = pa * pb
        gchunks.append(jax.lax.dot_general(
            onesblk_ref[...], p_ref[...], (((1,), (0,)), ((), ())),
            preferred_element_type=jnp.float32))

    def gram(a, b):
        idx = _PAIRS.index((min(a, b), max(a, b)))
        r = idx % _NUM_NODES
        return gchunks[idx // _NUM_NODES][r:r + 1, :]

    def whd(e, a):
        return wh[a][e:e + 1, :]

    def wwd(e, f):
        return ww[e:e + 1, f:f + 1]

    sc = []
    x2 = []
    for a in range(_NUM_NODES):
        n2 = gram(a, a)
        n = jnp.sqrt(n2)
        s = jnp.tanh(n) / (n + 1e-08)
        sc.append(s)
        x2.append(n2 * s * s)

    eid = 0
    for i in range(_NUM_NODES):
        ejs = [(eid + kk, j) for kk, j in enumerate(_NEIGH[i])]
        eid += len(ejs)
        d = len(ejs)
        D = {j: sc[i] * sc[j] * gram(i, j) for _, j in ejs}
        E = {e: sc[i] * whd(e, i) for e, _ in ejs}
        a2 = x2[i]
        coeffs = []
        for kk, (e, j) in enumerate(ejs):
            w2 = wwd(e, e)
            xw = sc[j] * whd(e, j)
            r = 1.0 / (1.0 + 2.0 * c * xw + (c * c) * x2[j] * w2 + 1e-08)
            ca = (1.0 + 2.0 * c * xw + c * w2) * r
            cb = (1.0 - c * x2[j]) * r
            t2 = ca * ca * x2[j] + 2.0 * ca * cb * xw + cb * cb * w2
            at = ca * D[j] + cb * E[e]
            rr = 1.0 / (1.0 + 2.0 * c * at + (c * c) * a2 * t2 + 1e-08)
            ga = (1.0 + 2.0 * c * at + c * t2) * rr
            gb = (1.0 - c * a2) * rr
            a2 = ga * ga * a2 + 2.0 * ga * gb * at + gb * gb * t2
            for e2, j2 in ejs[kk + 1:]:
                tdn = ca * sc[j] * sc[j2] * gram(j, j2) + cb * sc[j2] * whd(e, j2)
                D[j2] = ga * D[j2] + gb * tdn
                tdw = ca * sc[j] * whd(e2, j) + cb * wwd(e, e2)
                E[e2] = ga * E[e2] + gb * tdw
            coeffs.append((e, j, ca, cb, ga, gb))
        suf = [None] * d
        run = None
        for kk in range(d - 1, -1, -1):
            suf[kk] = run
            ga_k = coeffs[kk][4]
            run = ga_k if run is None else ga_k * run
        lam = run
        acc = (lam * sc[i]) * s_ref[pl.ds(i * _HIDDEN, _HIDDEN), :]
        for kk, (e, j, ca, cb, ga, gb) in enumerate(coeffs):
            s_k = gb if suf[kk] is None else gb * suf[kk]
            beta = s_k * ca * sc[j]
            gamma = s_k * cb
            i_e, j_e = _EDGES[e]
            wcol = mwt_ref[:, pl.ds(i_e * _NUM_NODES + j_e, 1)]
            acc = acc + beta * s_ref[pl.ds(j * _HIDDEN, _HIDDEN), :] \
                      + gamma * wcol
        out_ref[:, i, :] = jax.lax.dot_general(
            acc, wfrom_ref[...], (((0,), (1,)), ((), ())),
            preferred_element_type=jnp.float32) + bfrom


@functools.partial(jax.jit, static_argnames=("interpret",))
def kernel(node_features, W_to, b_to, W_from, b_from, curvature,
           mobius_weights, interpret=False):
    batch = node_features.shape[0]
    grid = batch // _B_TILE

    wrows = [mobius_weights[i, j] for i, j in _EDGES]
    wrows += [jnp.zeros((_HIDDEN,), jnp.float32)] * (_EPAD - _NUM_EDGES)
    wstack = jnp.stack(wrows, axis=0)

    onesblk = np.zeros((16, _NUM_NODES * _HIDDEN), dtype=np.float32)
    for r in range(_NUM_NODES):
        onesblk[r, r * _HIDDEN:(r + 1) * _HIDDEN] = 1.0

    full = lambda shape: pl.BlockSpec(shape, lambda b: (0,) * len(shape))
    out = pl.pallas_call(
        _body,
        grid=(grid,),
        in_specs=[
            pl.BlockSpec((_B_TILE, _NUM_NODES, _HIDDEN),
                         lambda b: (b, 0, 0)),
            full((_HIDDEN, _HIDDEN)),
            full((_HIDDEN, 1)),
            full((_HIDDEN, _HIDDEN)),
            full((1, _HIDDEN)),
            full((1, 1)),
            full((_HIDDEN, _NUM_NODES * _NUM_NODES)),
            full((_EPAD, _HIDDEN)),
            full((16, _NUM_NODES * _HIDDEN)),
        ],
        out_specs=pl.BlockSpec((_B_TILE, _NUM_NODES, _HIDDEN),
                               lambda b: (b, 0, 0)),
        out_shape=jax.ShapeDtypeStruct((batch, _NUM_NODES, _HIDDEN),
                                       jnp.float32),
        scratch_shapes=(
            [pltpu.VMEM((_NUM_NODES * _HIDDEN, _B_TILE), jnp.float32),
             pltpu.VMEM((_NUM_NODES * _HIDDEN, _B_TILE), jnp.float32)]
        ),
        interpret=interpret,
    )(
        node_features,
        W_to,
        b_to.reshape(_HIDDEN, 1),
        W_from,
        b_from.reshape(1, _HIDDEN),
        jnp.asarray(curvature, jnp.float32).reshape(1, 1),
        mobius_weights.reshape(_NUM_NODES * _NUM_NODES, _HIDDEN).T,
        wstack,
        jnp.asarray(onesblk),
    )
    return out
